# Initial kernel scaffold; baseline (speedup 1.0000x reference)
import jax
import jax.numpy as jnp
from jax.experimental import pallas as pl


def kernel(idx, weight):
    raise NotImplementedError("write your pallas kernel here")


# speedup vs baseline: 1.8286x; 1.8286x over previous
import functools

import jax
import jax.numpy as jnp
from jax import lax
from jax.experimental import pallas as pl
from jax.experimental.pallas import tpu as pltpu
from jax.experimental.pallas import tpu_sc as plsc

_NUM_WORKERS = 32
_CHUNK = 512


@functools.partial(jax.jit, static_argnames=("b_per_w", "n_chunks", "d"))
def _sc_embedding_lookup(idx_flat, weight, *, b_per_w, n_chunks, d):
    mesh = plsc.VectorSubcoreMesh(core_axis_name="c", subcore_axis_name="s")
    b_total = idx_flat.shape[0]

    @functools.partial(
        pl.kernel,
        mesh=mesh,
        out_type=jax.ShapeDtypeStruct((b_total, d), jnp.float32),
        scratch_types=[
            pltpu.VMEM((b_per_w,), jnp.int32),
            pltpu.VMEM((_CHUNK, d), jnp.float32),
            pltpu.SemaphoreType.DMA,
        ],
        compiler_params=pltpu.CompilerParams(use_tc_tiling_on_sc=False),
    )
    def k(idx_hbm, table_hbm, out_hbm, idx_v, rows_v, sem):
        nc = jax.lax.axis_size("c")
        wid = lax.axis_index("s") * nc + lax.axis_index("c")
        base = wid * b_per_w
        pltpu.sync_copy(idx_hbm.at[pl.ds(base, b_per_w)], idx_v)

        def body(j, carry):
            off = j * _CHUNK
            pltpu.async_copy(
                table_hbm.at[idx_v.at[pl.ds(off, _CHUNK)]], rows_v, sem
            ).wait()
            pltpu.sync_copy(rows_v, out_hbm.at[pl.ds(base + off, _CHUNK)])
            return carry

        lax.fori_loop(0, n_chunks, body, 0)

    return k(idx_flat, weight)


def kernel(idx, weight):
    b, s = idx.shape
    v, d = weight.shape
    b_total = b * s
    b_per_w = b_total // _NUM_WORKERS
    n_chunks = b_per_w // _CHUNK
    idx_flat = idx.reshape(b_total).astype(jnp.int32)
    out = _sc_embedding_lookup(
        idx_flat, weight, b_per_w=b_per_w, n_chunks=n_chunks, d=d
    )
    return out.reshape(b, s, d)


# speedup vs baseline: 1.8774x; 1.0267x over previous
import functools

import jax
import jax.numpy as jnp
from jax import lax
from jax.experimental import pallas as pl
from jax.experimental.pallas import tpu as pltpu
from jax.experimental.pallas import tpu_sc as plsc

_NUM_WORKERS = 32
_CHUNK = 256
_NBUF = 4


@functools.partial(jax.jit, static_argnames=("b_per_w", "n_chunks", "d"))
def _sc_embedding_lookup(idx_flat, weight, *, b_per_w, n_chunks, d):
    mesh = plsc.VectorSubcoreMesh(core_axis_name="c", subcore_axis_name="s")
    b_total = idx_flat.shape[0]

    @functools.partial(
        pl.kernel,
        mesh=mesh,
        out_type=jax.ShapeDtypeStruct((b_total, d), jnp.float32),
        scratch_types=[
            pltpu.VMEM((b_per_w,), jnp.int32),
            pltpu.VMEM((_NBUF, _CHUNK, d), jnp.float32),
            pltpu.SemaphoreType.DMA((_NBUF,)),
            pltpu.SemaphoreType.DMA((_NBUF,)),
        ],
        compiler_params=pltpu.CompilerParams(use_tc_tiling_on_sc=False),
    )
    def k(idx_hbm, table_hbm, out_hbm, idx_v, rows_v, gsem, wsem):
        nc = jax.lax.axis_size("c")
        wid = lax.axis_index("s") * nc + lax.axis_index("c")
        base = wid * b_per_w
        pltpu.sync_copy(idx_hbm.at[pl.ds(base, b_per_w)], idx_v)

        def start_gather(j, b):
            pltpu.make_async_copy(
                table_hbm.at[idx_v.at[pl.ds(j * _CHUNK, _CHUNK)]],
                rows_v.at[b],
                gsem.at[b],
            ).start()

        def wait_gather(b):
            pltpu.make_async_copy(
                table_hbm.at[idx_v.at[pl.ds(0, _CHUNK)]],
                rows_v.at[b],
                gsem.at[b],
            ).wait()

        def start_write(j, b):
            pltpu.make_async_copy(
                rows_v.at[b],
                out_hbm.at[pl.ds(base + j * _CHUNK, _CHUNK)],
                wsem.at[b],
            ).start()

        def wait_write(b):
            pltpu.make_async_copy(
                rows_v.at[b],
                out_hbm.at[pl.ds(base, _CHUNK)],
                wsem.at[b],
            ).wait()

        start_gather(0, 0)
        start_gather(1, 1)

        @pl.loop(0, n_chunks, step=_NBUF)
        def _outer(t):
            for i in range(_NBUF):
                j = t + i
                wait_gather(i)
                start_write(j, i)
                b2 = (i + 2) % _NBUF

                @pl.when(j >= 2)
                def _():
                    wait_write(b2)

                @pl.when(j + 2 < n_chunks)
                def _():
                    start_gather(j + 2, b2)

        wait_write((n_chunks - 2) % _NBUF)
        wait_write((n_chunks - 1) % _NBUF)

    return k(idx_flat, weight)


def kernel(idx, weight):
    b, s = idx.shape
    v, d = weight.shape
    b_total = b * s
    b_per_w = b_total // _NUM_WORKERS
    n_chunks = b_per_w // _CHUNK
    assert n_chunks % _NBUF == 0 and n_chunks >= 2 * _NBUF
    idx_flat = idx.reshape(b_total).astype(jnp.int32)
    out = _sc_embedding_lookup(
        idx_flat, weight, b_per_w=b_per_w, n_chunks=n_chunks, d=d
    )
    return out.reshape(b, s, d)
